# Initial kernel scaffold; baseline (speedup 1.0000x reference)
import jax
import jax.numpy as jnp
from jax.experimental import pallas as pl


def kernel(x, edge_index, edge_attr, batch, W1, b1, g1, be1, W2, b2, g2, be2, W3, b3, g3, be3, Wc, bc):
    raise NotImplementedError("write your pallas kernel here")


# speedup vs baseline: 6.0516x; 6.0516x over previous
import functools

import jax
import jax.numpy as jnp
from jax import lax
from jax.experimental import pallas as pl
from jax.experimental.pallas import tpu as pltpu
from jax.experimental.pallas import tpu_sc as plsc

_N = 10000
_E = 320000
_FIN = 128
_H = 256
_NB = 4
_HB = _H // _NB
_C = 40
_G = 64
_NC = 2
_NS = 16
_NW = _NC * _NS
_EPW = _E // _NW
_ET = _E // _NS
_B = 80
_NCHUNK = _ET // _B
_RPT = _N // _NS
_ZR = 125
_EPS = 1e-5

_sc_mesh = plsc.VectorSubcoreMesh(core_axis_name="c", subcore_axis_name="s")
_sc_params = pltpu.CompilerParams(use_tc_tiling_on_sc=False)


@functools.partial(
    pl.kernel,
    out_type=jax.ShapeDtypeStruct((_NW, _N // 16, 16), jnp.float32),
    mesh=_sc_mesh,
    compiler_params=_sc_params,
    scratch_types=[
        pltpu.VMEM((_EPW // 16, 16), jnp.int32),
        pltpu.VMEM((_EPW // 16, 16), jnp.float32),
        pltpu.VMEM((_N // 16, 16), jnp.float32),
    ],
)
def _deg_kernel(col_hbm, ew_hbm, out_hbm, col_v, ew_v, deg_v):
    c = lax.axis_index("c")
    s = lax.axis_index("s")
    wid = s * _NC + c
    pltpu.sync_copy(col_hbm.at[wid], col_v)
    pltpu.sync_copy(ew_hbm.at[wid], ew_v)

    def zero(i, carry):
        deg_v[i] = jnp.zeros((16,), jnp.float32)
        return carry

    lax.fori_loop(0, _N // 16, zero, 0)
    lanes = lax.iota(jnp.int32, 16)

    def grp(g, carry):
        cw = col_v[g]
        ww = ew_v[g]
        for i in range(16):
            ci = cw[i]
            r = lax.shift_right_logical(ci, 4)
            lane = lax.bitwise_and(ci, 15)
            deg_v[r] = deg_v[r] + jnp.where(lanes == lane, ww[i], 0.0)
        return carry

    lax.fori_loop(0, _EPW // 16, grp, 0)
    pltpu.sync_copy(deg_v, out_hbm.at[wid])


@functools.partial(
    pl.kernel,
    out_type=jax.ShapeDtypeStruct((_NB * _N, _HB), jnp.float32),
    mesh=_sc_mesh,
    compiler_params=_sc_params,
    scratch_types=[
        pltpu.VMEM((_NCHUNK, _B), jnp.int32),
        pltpu.VMEM((_NCHUNK, _B), jnp.int32),
        pltpu.VMEM((_ET // 16, 16), jnp.float32),
        pltpu.VMEM((_B, _HB), jnp.float32),
        pltpu.VMEM((_ZR, _HB), jnp.float32),
        pltpu.VMEM_SHARED((_N, _HB), jnp.float32),
        pltpu.SemaphoreType.DMA,
    ],
)
def _agg_kernel(hd_hbm, row_hbm, col_hbm, ew_hbm, out_hbm,
                idx_v, col_v, ew_v, rows_v, zero_v, acc_sh, sem):
    c = lax.axis_index("c")
    s = lax.axis_index("s")
    pltpu.sync_copy(row_hbm.at[s], idx_v)
    pltpu.sync_copy(col_hbm.at[s], col_v)
    pltpu.sync_copy(ew_hbm.at[s], ew_v)

    def zfill(i, carry):
        for t in range(_HB // 16):
            zero_v[i, pl.ds(t * 16, 16)] = jnp.zeros((16,), jnp.float32)
        return carry

    lax.fori_loop(0, _ZR, zfill, 0)

    def shift(delta):
        def adjust(j, carry):
            for t in range(_B // 16):
                sl = pl.ds(t * 16, 16)
                idx_v[j, sl] = idx_v[j, sl] + delta
            return carry

        lax.fori_loop(0, _NCHUNK, adjust, 0)

    shift(c * _N)

    for p in range(2):
        if p:
            shift(2 * _N)
        blk = 2 * p + c
        for r in range(_RPT // _ZR):
            pltpu.sync_copy(zero_v, acc_sh.at[pl.ds(s * _RPT + r * _ZR, _ZR)])
        plsc.subcore_barrier()

        def chunk(j, carry):
            pltpu.async_copy(hd_hbm.at[idx_v.at[j]], rows_v, sem).wait()

            def grp(kb, carry2):
                w16 = ew_v[j * (_B // 16) + kb]
                for i in range(16):
                    w = w16[i]
                    k = kb * 16 + i
                    for t in range(_HB // 16):
                        sl = pl.ds(t * 16, 16)
                        rows_v[k, sl] = rows_v[k, sl] * w
                return carry2

            lax.fori_loop(0, _B // 16, grp, 0)
            pltpu.sync_copy(rows_v, acc_sh.at[col_v.at[j]], add=True)
            return carry

        lax.fori_loop(0, _NCHUNK, chunk, 0)
        plsc.subcore_barrier()
        pltpu.sync_copy(acc_sh.at[pl.ds(s * _RPT, _RPT)],
                        out_hbm.at[pl.ds(blk * _N + s * _RPT, _RPT)])


def _prep_body(degp_ref, x_ref, w1_ref, hd_ref, dis_ref):
    deg = jnp.sum(degp_ref[...], axis=0) + 1.0
    dis = lax.rsqrt(deg).reshape(_N, 1)
    h = jnp.dot(x_ref[...], w1_ref[...], preferred_element_type=jnp.float32)
    hd = h * dis
    for b in range(_NB):
        hd_ref[b] = hd[:, b * _HB:(b + 1) * _HB]
    dis_ref[...] = dis


_prep = pl.pallas_call(
    _prep_body,
    out_shape=[
        jax.ShapeDtypeStruct((_NB, _N, _HB), jnp.float32),
        jax.ShapeDtypeStruct((_N, 1), jnp.float32),
    ],
)


_BLK = 1000
_NSTEP = _N // _BLK


def _t_block(agg_ref, hd_ref, dis_ref, b_ref):
    dis = dis_ref[...]
    cols = []
    for b in range(_NB):
        sl = slice(b * _HB, (b + 1) * _HB)
        cols.append((agg_ref[b] + hd_ref[b]) * dis + b_ref[:, sl])
    return jnp.concatenate(cols, axis=1)


def _stats_body(agg_ref, hd_ref, dis_ref, b_ref, o_ref):
    i = pl.program_id(0)
    t = _t_block(agg_ref, hd_ref, dis_ref, b_ref)
    s = jnp.sum(t, axis=0, keepdims=True)
    q = jnp.sum(t * t, axis=0, keepdims=True)
    sq = jnp.concatenate([s, q], axis=0)

    @pl.when(i == 0)
    def _():
        o_ref[...] = sq

    @pl.when(i > 0)
    def _():
        o_ref[...] += sq


_stats = pl.pallas_call(
    _stats_body,
    grid=(_NSTEP,),
    in_specs=[
        pl.BlockSpec((_NB, _BLK, _HB), lambda i: (0, i, 0)),
        pl.BlockSpec((_NB, _BLK, _HB), lambda i: (0, i, 0)),
        pl.BlockSpec((_BLK, 1), lambda i: (i, 0)),
        pl.BlockSpec((1, _H), lambda i: (0, 0)),
    ],
    out_specs=pl.BlockSpec((2, _H), lambda i: (0, 0)),
    out_shape=jax.ShapeDtypeStruct((2, _H), jnp.float32),
)


def _bn_relu(t, sq_ref, g_ref, be_ref):
    mu = sq_ref[0:1, :] * (1.0 / _N)
    var = sq_ref[1:2, :] * (1.0 / _N) - mu * mu
    alpha = lax.rsqrt(var + _EPS) * g_ref[...]
    return jnp.maximum((t - mu) * alpha + be_ref[...], 0.0)


def _apply_body(agg_ref, hd_ref, dis_ref, b_ref, g_ref, be_ref, sq_ref,
                w_ref, o_ref):
    t = _t_block(agg_ref, hd_ref, dis_ref, b_ref)
    u = _bn_relu(t, sq_ref, g_ref, be_ref)
    hn = jnp.dot(u, w_ref[...], preferred_element_type=jnp.float32)
    dis = dis_ref[...]
    for b in range(_NB):
        o_ref[b] = hn[:, b * _HB:(b + 1) * _HB] * dis


_apply = pl.pallas_call(
    _apply_body,
    grid=(_NSTEP,),
    in_specs=[
        pl.BlockSpec((_NB, _BLK, _HB), lambda i: (0, i, 0)),
        pl.BlockSpec((_NB, _BLK, _HB), lambda i: (0, i, 0)),
        pl.BlockSpec((_BLK, 1), lambda i: (i, 0)),
        pl.BlockSpec((1, _H), lambda i: (0, 0)),
        pl.BlockSpec((1, _H), lambda i: (0, 0)),
        pl.BlockSpec((1, _H), lambda i: (0, 0)),
        pl.BlockSpec((2, _H), lambda i: (0, 0)),
        pl.BlockSpec((_H, _H), lambda i: (0, 0)),
    ],
    out_specs=pl.BlockSpec((_NB, _BLK, _HB), lambda i: (0, i, 0)),
    out_shape=jax.ShapeDtypeStruct((_NB, _N, _HB), jnp.float32),
)


def _pool_body(agg_ref, hd_ref, dis_ref, b_ref, g_ref, be_ref, sq_ref,
               batch_ref, wc_ref, bc_ref, o_ref, acc_s, cnt_s):
    i = pl.program_id(0)
    t = _t_block(agg_ref, hd_ref, dis_ref, b_ref)
    u = _bn_relu(t, sq_ref, g_ref, be_ref)
    oneh = (batch_ref[0] == lax.broadcasted_iota(jnp.int32, (_G, _BLK), 0))
    oneh = oneh.astype(jnp.float32)
    sums = jnp.dot(oneh, u, preferred_element_type=jnp.float32)
    cnts = jnp.dot(oneh, jnp.ones((_BLK, 1), jnp.float32),
                   preferred_element_type=jnp.float32)

    @pl.when(i == 0)
    def _():
        acc_s[...] = sums
        cnt_s[...] = cnts

    @pl.when(i > 0)
    def _():
        acc_s[...] += sums
        cnt_s[...] += cnts

    @pl.when(i == _NSTEP - 1)
    def _():
        pooled = acc_s[...] / jnp.maximum(cnt_s[...], 1.0)
        logits = jnp.dot(pooled, wc_ref[...],
                         preferred_element_type=jnp.float32) + bc_ref[...]
        z = logits - jnp.max(logits, axis=1, keepdims=True)
        o_ref[...] = z - jnp.log(jnp.sum(jnp.exp(z), axis=1, keepdims=True))


_pool = pl.pallas_call(
    _pool_body,
    grid=(_NSTEP,),
    in_specs=[
        pl.BlockSpec((_NB, _BLK, _HB), lambda i: (0, i, 0)),
        pl.BlockSpec((_NB, _BLK, _HB), lambda i: (0, i, 0)),
        pl.BlockSpec((_BLK, 1), lambda i: (i, 0)),
        pl.BlockSpec((1, _H), lambda i: (0, 0)),
        pl.BlockSpec((1, _H), lambda i: (0, 0)),
        pl.BlockSpec((1, _H), lambda i: (0, 0)),
        pl.BlockSpec((2, _H), lambda i: (0, 0)),
        pl.BlockSpec((1, 1, _BLK), lambda i: (i, 0, 0)),
        pl.BlockSpec((_H, _C), lambda i: (0, 0)),
        pl.BlockSpec((1, _C), lambda i: (0, 0)),
    ],
    out_specs=pl.BlockSpec((_G, _C), lambda i: (0, 0)),
    out_shape=jax.ShapeDtypeStruct((_G, _C), jnp.float32),
    scratch_shapes=[
        pltpu.VMEM((_G, _H), jnp.float32),
        pltpu.VMEM((_G, 1), jnp.float32),
    ],
)


def kernel(x, edge_index, edge_attr, batch,
           W1, b1, g1, be1, W2, b2, g2, be2, W3, b3, g3, be3, Wc, bc):
    row = edge_index[0]
    col = edge_index[1]
    ew = edge_attr.reshape(_E).astype(jnp.float32)

    degp = _deg_kernel(col.reshape(_NW, _EPW // 16, 16),
                       ew.reshape(_NW, _EPW // 16, 16))
    hd1, dis = _prep(degp.reshape(_NW, _N), x, W1)

    row3 = row.reshape(_NS, _NCHUNK, _B)
    col3 = col.reshape(_NS, _NCHUNK, _B)
    ew3 = ew.reshape(_NS, _ET // 16, 16)

    def agg(hd):
        out = _agg_kernel(hd.reshape(_NB * _N, _HB), row3, col3, ew3)
        return out.reshape(_NB, _N, _HB)

    b1r, g1r, be1r = b1.reshape(1, _H), g1.reshape(1, _H), be1.reshape(1, _H)
    b2r, g2r, be2r = b2.reshape(1, _H), g2.reshape(1, _H), be2.reshape(1, _H)
    b3r, g3r, be3r = b3.reshape(1, _H), g3.reshape(1, _H), be3.reshape(1, _H)

    agg1 = agg(hd1)
    sq1 = _stats(agg1, hd1, dis, b1r)
    hd2 = _apply(agg1, hd1, dis, b1r, g1r, be1r, sq1, W2)
    agg2 = agg(hd2)
    sq2 = _stats(agg2, hd2, dis, b2r)
    hd3 = _apply(agg2, hd2, dis, b2r, g2r, be2r, sq2, W3)
    agg3 = agg(hd3)
    sq3 = _stats(agg3, hd3, dis, b3r)
    return _pool(agg3, hd3, dis, b3r, g3r, be3r, sq3,
                 batch.reshape(_NSTEP, 1, _BLK), Wc, bc.reshape(1, _C))
